# baseline (speedup 1.0000x reference)
import functools

import jax
import jax.numpy as jnp
from jax.experimental import pallas as pl
from jax.experimental.pallas import tpu as pltpu


def _mlp_kernel(xa_ref, xb_ref, w1t_ref, b1_ref, w2_ref, b2_ref, o_ref):
    smask = jax.lax.broadcasted_iota(jnp.int32, (8, 128), 0)
    b2v = b2_ref[0]
    w1t = w1t_ref[...]
    b1c = b1_ref[...]
    w2c = w2_ref[...]
    half_rows = xa_ref.shape[1] // 128
    for part, x_ref in enumerate((xa_ref, xb_ref)):
        base = part * half_rows
        for q in range(x_ref.shape[1] // 1024):
            xq = x_ref[:, q * 1024:(q + 1) * 1024]
            z = jnp.dot(w1t, xq, preferred_element_type=jnp.float32)
            hw = jnp.maximum(z + b1c, 0.0) * w2c
            t = hw[0:8] + hw[8:16] + hw[16:24] + hw[24:32]
            acc = jnp.zeros((8, 128), jnp.float32)
            for j in range(8):
                s = jnp.sum(t[:, j * 128:(j + 1) * 128], axis=0, keepdims=True)
                acc = acc + jnp.where(smask == j, s, 0.0)
            o_ref[base + q * 8:base + (q + 1) * 8, :] = acc + b2v


@functools.partial(jax.jit, static_argnames=("tile_obs",))
def _value_net_forward(x, w1, b1, w2, b2, *, tile_obs=131072):
    x = jnp.asarray(x, jnp.float32)
    B, in_size = x.shape
    hidden = w1.shape[1]

    num_tiles = pl.cdiv(B, tile_obs)
    if num_tiles > 1:
        num_tiles = ((num_tiles + 1) // 2) * 2
    b_pad = num_tiles * tile_obs

    xt = x.T
    if b_pad != B:
        xt = jnp.pad(xt, ((0, 0), (0, b_pad - B)))

    w1t = w1.astype(jnp.float32).T
    b1c = b1.astype(jnp.float32).reshape(hidden, 1)
    w2c = w2.astype(jnp.float32).reshape(hidden, 1)
    b2_s = b2.reshape(1).astype(jnp.float32)

    flops = 2 * b_pad * (in_size * hidden + hidden)
    bytes_accessed = 4 * (xt.size + w1t.size + hidden * 2 + 1 + b_pad)

    out = pl.pallas_call(
        _mlp_kernel,
        out_shape=jax.ShapeDtypeStruct((b_pad // 128, 128), jnp.float32),
        grid=(num_tiles,),
        in_specs=[
            pl.BlockSpec((in_size, tile_obs // 2),
                         lambda i: (0, 2 * i)),
            pl.BlockSpec((in_size, tile_obs // 2),
                         lambda i: (0, 2 * i + 1)),
            pl.BlockSpec((hidden, in_size), lambda i: (0, 0)),
            pl.BlockSpec((hidden, 1), lambda i: (0, 0)),
            pl.BlockSpec((hidden, 1), lambda i: (0, 0)),
            pl.BlockSpec(memory_space=pltpu.MemorySpace.SMEM),
        ],
        out_specs=pl.BlockSpec((tile_obs // 128, 128), lambda i: (i, 0)),
        compiler_params=pltpu.CompilerParams(
            dimension_semantics=("parallel",),
            vmem_limit_bytes=64 * 1024 * 1024,
        ),
        cost_estimate=pl.CostEstimate(
            flops=flops, transcendentals=0, bytes_accessed=bytes_accessed),
    )(xt, xt, w1t, b1c, w2c, b2_s)

    return out.reshape(b_pad, 1)[:B]


def kernel(x, w1, b1, w2, b2):
    return _value_net_forward(x, w1, b1, w2, b2)


# speedup vs baseline: 1.0692x; 1.0692x over previous
import functools

import jax
import jax.numpy as jnp
from jax.experimental import pallas as pl
from jax.experimental.pallas import tpu as pltpu


def _mlp_kernel(xt_ref, w1_ref, bw_ref, b2_ref, o_ref):
    T = xt_ref.shape[1]
    smask = jax.lax.broadcasted_iota(jnp.int32, (8, 128), 0)
    b2v = b2_ref[0]
    w1t = w1_ref[...].T
    bw = bw_ref[...].T
    b1c = bw[:, 0:1]
    w2c = bw[:, 1:2]
    for q in range(T // 1024):
        xq = xt_ref[:, q * 1024:(q + 1) * 1024]
        z = jnp.dot(w1t, xq, preferred_element_type=jnp.float32)
        hw = jnp.maximum(z + b1c, 0.0) * w2c
        t = hw[0:8] + hw[8:16] + hw[16:24] + hw[24:32]
        acc = jnp.zeros((8, 128), jnp.float32)
        for j in range(8):
            s = jnp.sum(t[:, j * 128:(j + 1) * 128], axis=0, keepdims=True)
            acc = acc + jnp.where(smask == j, s, 0.0)
        o_ref[q * 8:(q + 1) * 8, :] = acc + b2v


@functools.partial(jax.jit, static_argnames=("tile_obs",))
def _value_net_forward(x, w1, b1, w2, b2, *, tile_obs=131072):
    x = jnp.asarray(x, jnp.float32)
    B, in_size = x.shape
    hidden = w1.shape[1]

    num_tiles = pl.cdiv(B, tile_obs)
    if num_tiles > 1:
        num_tiles = ((num_tiles + 1) // 2) * 2
    b_pad = num_tiles * tile_obs

    xt = x.T
    if b_pad != B:
        xt = jnp.pad(xt, ((0, 0), (0, b_pad - B)))

    w1f = w1.astype(jnp.float32)
    bw = jnp.stack([b1.astype(jnp.float32),
                    w2.astype(jnp.float32).reshape(-1)])
    b2_s = b2.reshape(1).astype(jnp.float32)

    flops = 2 * b_pad * (in_size * hidden + hidden)
    bytes_accessed = 4 * (xt.size + w1f.size + hidden * 2 + 1 + b_pad)

    out = pl.pallas_call(
        _mlp_kernel,
        out_shape=jax.ShapeDtypeStruct((b_pad // 128, 128), jnp.float32),
        grid=(num_tiles,),
        in_specs=[
            pl.BlockSpec((in_size, tile_obs), lambda i: (0, i)),
            pl.BlockSpec((in_size, hidden), lambda i: (0, 0)),
            pl.BlockSpec((2, hidden), lambda i: (0, 0)),
            pl.BlockSpec(memory_space=pltpu.MemorySpace.SMEM),
        ],
        out_specs=pl.BlockSpec((tile_obs // 128, 128), lambda i: (i, 0)),
        compiler_params=pltpu.CompilerParams(
            dimension_semantics=("parallel",),
            vmem_limit_bytes=64 * 1024 * 1024,
        ),
        cost_estimate=pl.CostEstimate(
            flops=flops, transcendentals=0, bytes_accessed=bytes_accessed),
    )(xt, w1f, bw, b2_s)

    return out.reshape(b_pad, 1)[:B]


def kernel(x, w1, b1, w2, b2):
    return _value_net_forward(x, w1, b1, w2, b2)
